# baseline (speedup 1.0000x reference)
import jax
import jax.numpy as jnp
from jax import lax
from jax.experimental import pallas as pl
from jax.experimental.pallas import tpu as pltpu


def _chamfer_body(srcT_ref, tgtT_ref, out_ref):
    b = pl.program_id(0)
    nb = pl.num_programs(0)
    srcT_aug = srcT_ref[0]
    tgt_aug = tgtT_ref[0]

    d2 = lax.dot_general(
        srcT_aug, tgt_aug, (((0,), (0,)), ((), ())),
        preferred_element_type=jnp.float32,
    )

    rowmin = jnp.min(d2, axis=1, keepdims=True)
    colmin = jnp.min(d2, axis=0, keepdims=True)

    n = srcT_aug.shape[1]
    m = tgt_aug.shape[1]
    batch_val = (
        jnp.sum(jnp.maximum(rowmin, 0.0)) / n
        + jnp.sum(jnp.maximum(colmin, 0.0)) / m
    )

    @pl.when(b == 0)
    def _():
        out_ref[0, 0] = 0.0

    out_ref[0, 0] += batch_val / nb


@jax.jit
def kernel(src_points, tgt_points):
    B, N, D = src_points.shape
    M = tgt_points.shape[1]

    sq_s = jnp.sum(src_points * src_points, axis=-1, keepdims=True)
    sq_t = jnp.sum(tgt_points * tgt_points, axis=-1, keepdims=True)
    ones_s = jnp.ones((B, N, 1), jnp.float32)
    ones_t = jnp.ones((B, M, 1), jnp.float32)
    src_aug = jnp.concatenate(
        [-2.0 * src_points, ones_s, sq_s, jnp.zeros((B, N, 3), jnp.float32)], axis=-1
    )
    tgt_aug = jnp.concatenate(
        [tgt_points, sq_t, ones_t, jnp.zeros((B, M, 3), jnp.float32)], axis=-1
    )
    all_aug = jnp.transpose(jnp.concatenate([src_aug, tgt_aug], axis=1), (0, 2, 1))

    out = pl.pallas_call(
        _chamfer_body,
        grid=(B,),
        in_specs=[
            pl.BlockSpec((1, 8, N), lambda b: (b, 0, 0)),
            pl.BlockSpec((1, 8, M), lambda b: (b, 0, 1)),
        ],
        out_specs=pl.BlockSpec((1, 1), lambda b: (0, 0), memory_space=pltpu.SMEM),
        out_shape=jax.ShapeDtypeStruct((1, 1), jnp.float32),
    )(all_aug, all_aug)
    return out[0, 0]
